# Initial kernel scaffold; baseline (speedup 1.0000x reference)
import jax
import jax.numpy as jnp
from jax.experimental import pallas as pl


def kernel(z, cov_embedding, wearable_embedding, expert_Ws, expert_bs, expert_Wt, expert_bt, expert_W1, expert_b1, expert_W2, expert_b2, router_W1, router_b1, router_W2, router_b2):
    raise NotImplementedError("write your pallas kernel here")


# speedup vs baseline: 2.3129x; 2.3129x over previous
import functools

import jax
import jax.numpy as jnp
from jax.experimental import pallas as pl
from jax.experimental.pallas import tpu as pltpu

K = 25
PAD = (K - 1) // 2
NE = 8
B, L, P = 1024, 1024, 1024
HID = 64
UW = 0.3


def _prep_kernel(z_ref, comb_ref, rw1_ref, rb1_ref, rw2_ref, rb2_ref,
                 res_ref, mm_ref, wvec_ref):
    Z = z_ref[...]
    f32 = jnp.float32
    dn = (((1,), (1,)), ((), ()))
    jc = jax.lax.broadcasted_iota(jnp.int32, (L, L), 0).astype(f32)
    lc = jax.lax.broadcasted_iota(jnp.int32, (L, L), 1).astype(f32)
    band = (jnp.abs(jc - lc) <= PAD).astype(f32)
    front = jnp.where(lc == 0, jnp.maximum(PAD - jc, 0.0), 0.0)
    back = jnp.where(lc == L - 1, jnp.maximum(jc - (L - 1 - PAD), 0.0), 0.0)
    A = (band + front + back) * (1.0 / K)
    mm = jax.lax.dot_general(Z, A, dn, preferred_element_type=f32)
    mm_ref[...] = mm
    res_ref[...] = Z - mm
    h = jnp.maximum(
        jax.lax.dot_general(comb_ref[...], rw1_ref[...], dn,
                            preferred_element_type=jnp.float32) + rb1_ref[...],
        0.0)
    logits = jax.lax.dot_general(h, rw2_ref[...], dn,
                                 preferred_element_type=jnp.float32) + rb2_ref[...]
    mx = jnp.max(logits, axis=1, keepdims=True)
    e = jnp.exp(logits - mx)
    sm = e / jnp.sum(e, axis=1, keepdims=True) * (1.0 - UW)
    wvec_ref[...] = jnp.concatenate(
        [jnp.full((B, 1), UW, jnp.float32), sm[:, :NE - 1]], axis=1)


def _expert_kernel(res_ref, mm_ref, wvec_ref, ws_ref, wt_ref, bst_ref,
                   w1_ref, b1_ref, w2_ref, b2_ref, out_ref):
    i = pl.program_id(0)
    dn = (((1,), (1,)), ((), ()))
    f32 = jnp.float32
    dec = (jax.lax.dot_general(res_ref[...], ws_ref[0], dn,
                               preferred_element_type=f32)
           + jax.lax.dot_general(mm_ref[...], wt_ref[0], dn,
                                 preferred_element_type=f32)
           + bst_ref[0])
    h = jnp.maximum(
        jax.lax.dot_general(dec, w1_ref[0], dn, preferred_element_type=f32)
        + b1_ref[0], 0.0)
    o = jax.lax.dot_general(h, w2_ref[0], dn, preferred_element_type=f32) \
        + b2_ref[0]
    lane = jax.lax.broadcasted_iota(jnp.int32, (1, NE), 1)
    w = jnp.sum(wvec_ref[...] * (lane == i).astype(f32), axis=1, keepdims=True)
    contrib = w * o

    @pl.when(i == 0)
    def _():
        out_ref[...] = contrib

    @pl.when(i > 0)
    def _():
        out_ref[...] += contrib


@functools.partial(jax.jit, static_argnames=())
def kernel(z, cov_embedding, wearable_embedding, expert_Ws, expert_bs,
           expert_Wt, expert_bt, expert_W1, expert_b1, expert_W2, expert_b2,
           router_W1, router_b1, router_W2, router_b2):
    zsq = z[:, :, 0]
    comb = jnp.concatenate([cov_embedding, wearable_embedding], axis=1)
    rb1 = router_b1.reshape(1, HID)
    rw2 = jnp.concatenate([router_W2, jnp.zeros((1, HID), jnp.float32)], axis=0)
    rb2 = jnp.concatenate([router_b2, jnp.full((1,), -1e9, jnp.float32)]
                          ).reshape(1, NE)
    bst = (expert_bs + expert_bt).reshape(NE, 1, P)
    b1 = expert_b1.reshape(NE, 1, HID)
    b2 = expert_b2.reshape(NE, 1, P)

    res, mm, wvec = pl.pallas_call(
        _prep_kernel,
        out_shape=(
            jax.ShapeDtypeStruct((B, L), jnp.float32),
            jax.ShapeDtypeStruct((B, L), jnp.float32),
            jax.ShapeDtypeStruct((B, NE), jnp.float32),
        ),
    )(zsq, comb, router_W1, rb1, rw2, rb2)

    out = pl.pallas_call(
        _expert_kernel,
        grid=(NE,),
        in_specs=[
            pl.BlockSpec((B, L), lambda i: (0, 0)),
            pl.BlockSpec((B, L), lambda i: (0, 0)),
            pl.BlockSpec((B, NE), lambda i: (0, 0)),
            pl.BlockSpec((1, P, L), lambda i: (i, 0, 0)),
            pl.BlockSpec((1, P, L), lambda i: (i, 0, 0)),
            pl.BlockSpec((1, 1, P), lambda i: (i, 0, 0)),
            pl.BlockSpec((1, HID, P), lambda i: (i, 0, 0)),
            pl.BlockSpec((1, 1, HID), lambda i: (i, 0, 0)),
            pl.BlockSpec((1, P, HID), lambda i: (i, 0, 0)),
            pl.BlockSpec((1, 1, P), lambda i: (i, 0, 0)),
        ],
        out_specs=pl.BlockSpec((B, P), lambda i: (0, 0)),
        out_shape=jax.ShapeDtypeStruct((B, P), jnp.float32),
    )(res, mm, wvec, expert_Ws, expert_Wt, bst, expert_W1, b1, expert_W2, b2)

    return out[..., None]


# speedup vs baseline: 2.3675x; 1.0236x over previous
import functools

import jax
import jax.numpy as jnp
from jax.experimental import pallas as pl
from jax.experimental.pallas import tpu as pltpu

K = 25
PAD = (K - 1) // 2
NE = 8
B, L, P = 1024, 1024, 1024
HID = 64
UW = 0.3


def _prep_kernel(z_ref, comb_ref, rw1_ref, rb1_ref, rw2_ref, rb2_ref,
                 res_ref, mm_ref, wvec_ref):
    Z = z_ref[...]
    f32 = jnp.float32
    dn = (((1,), (1,)), ((), ()))
    jc = jax.lax.broadcasted_iota(jnp.int32, (L, L), 0).astype(f32)
    lc = jax.lax.broadcasted_iota(jnp.int32, (L, L), 1).astype(f32)
    band = (jnp.abs(jc - lc) <= PAD).astype(f32)
    front = jnp.where(lc == 0, jnp.maximum(PAD - jc, 0.0), 0.0)
    back = jnp.where(lc == L - 1, jnp.maximum(jc - (L - 1 - PAD), 0.0), 0.0)
    A = ((band + front + back) * (1.0 / K)).astype(jnp.bfloat16)
    mm = jax.lax.dot_general(Z.astype(jnp.bfloat16), A, dn,
                             preferred_element_type=f32)
    mm_ref[...] = mm.astype(jnp.bfloat16)
    res_ref[...] = (Z - mm).astype(jnp.bfloat16)
    h = jnp.maximum(
        jax.lax.dot_general(comb_ref[...], rw1_ref[...], dn,
                            preferred_element_type=jnp.float32) + rb1_ref[...],
        0.0)
    logits = jax.lax.dot_general(h, rw2_ref[...], dn,
                                 preferred_element_type=jnp.float32) + rb2_ref[...]
    mx = jnp.max(logits, axis=1, keepdims=True)
    e = jnp.exp(logits - mx)
    sm = e / jnp.sum(e, axis=1, keepdims=True) * (1.0 - UW)
    wvec_ref[...] = jnp.concatenate(
        [jnp.full((B, 1), UW, jnp.float32), sm[:, :NE - 1]], axis=1)


def _expert_kernel(res_ref, mm_ref, wvec_ref, ws_ref, wt_ref, bst_ref,
                   w1_ref, b1_ref, w2_ref, b2_ref, out_ref):
    i = pl.program_id(0)
    dn = (((1,), (1,)), ((), ()))
    f32 = jnp.float32
    bf16 = jnp.bfloat16
    dec = (jax.lax.dot_general(res_ref[...], ws_ref[0].astype(bf16), dn,
                               preferred_element_type=f32)
           + jax.lax.dot_general(mm_ref[...], wt_ref[0].astype(bf16), dn,
                                 preferred_element_type=f32)
           + bst_ref[0])
    h = jnp.maximum(
        jax.lax.dot_general(dec.astype(bf16), w1_ref[0].astype(bf16), dn,
                            preferred_element_type=f32)
        + b1_ref[0], 0.0)
    o = jax.lax.dot_general(h.astype(bf16), w2_ref[0].astype(bf16), dn,
                            preferred_element_type=f32) \
        + b2_ref[0]
    lane = jax.lax.broadcasted_iota(jnp.int32, (1, NE), 1)
    w = jnp.sum(wvec_ref[...] * (lane == i).astype(f32), axis=1, keepdims=True)
    contrib = w * o

    @pl.when(i == 0)
    def _():
        out_ref[...] = contrib

    @pl.when(i > 0)
    def _():
        out_ref[...] += contrib


@functools.partial(jax.jit, static_argnames=())
def kernel(z, cov_embedding, wearable_embedding, expert_Ws, expert_bs,
           expert_Wt, expert_bt, expert_W1, expert_b1, expert_W2, expert_b2,
           router_W1, router_b1, router_W2, router_b2):
    zsq = z[:, :, 0]
    comb = jnp.concatenate([cov_embedding, wearable_embedding], axis=1)
    rb1 = router_b1.reshape(1, HID)
    rw2 = jnp.concatenate([router_W2, jnp.zeros((1, HID), jnp.float32)], axis=0)
    rb2 = jnp.concatenate([router_b2, jnp.full((1,), -1e9, jnp.float32)]
                          ).reshape(1, NE)
    bst = (expert_bs + expert_bt).reshape(NE, 1, P)
    b1 = expert_b1.reshape(NE, 1, HID)
    b2 = expert_b2.reshape(NE, 1, P)

    res, mm, wvec = pl.pallas_call(
        _prep_kernel,
        out_shape=(
            jax.ShapeDtypeStruct((B, L), jnp.bfloat16),
            jax.ShapeDtypeStruct((B, L), jnp.bfloat16),
            jax.ShapeDtypeStruct((B, NE), jnp.float32),
        ),
    )(zsq, comb, router_W1, rb1, rw2, rb2)

    out = pl.pallas_call(
        _expert_kernel,
        grid=(NE,),
        in_specs=[
            pl.BlockSpec((B, L), lambda i: (0, 0)),
            pl.BlockSpec((B, L), lambda i: (0, 0)),
            pl.BlockSpec((B, NE), lambda i: (0, 0)),
            pl.BlockSpec((1, P, L), lambda i: (i, 0, 0)),
            pl.BlockSpec((1, P, L), lambda i: (i, 0, 0)),
            pl.BlockSpec((1, 1, P), lambda i: (i, 0, 0)),
            pl.BlockSpec((1, HID, P), lambda i: (i, 0, 0)),
            pl.BlockSpec((1, 1, HID), lambda i: (i, 0, 0)),
            pl.BlockSpec((1, P, HID), lambda i: (i, 0, 0)),
            pl.BlockSpec((1, 1, P), lambda i: (i, 0, 0)),
        ],
        out_specs=pl.BlockSpec((B, P), lambda i: (0, 0)),
        out_shape=jax.ShapeDtypeStruct((B, P), jnp.float32),
    )(res, mm, wvec, expert_Ws, expert_Wt, bst, expert_W1, b1, expert_W2, b2)

    return out[..., None]


# speedup vs baseline: 2.4787x; 1.0470x over previous
import functools

import jax
import jax.numpy as jnp
from jax.experimental import pallas as pl
from jax.experimental.pallas import tpu as pltpu

K = 25
PAD = (K - 1) // 2
NE = 8
B, L, P = 1024, 1024, 1024
HID = 64
UW = 0.3


def _moe_kernel(z_ref, comb_ref, rw1_ref, rb1_ref, rw2_ref, rb2_ref,
                bst_ref, b1_ref, b2_ref, ws_ref, wt_ref, w1_ref,
                w2_ref, out_ref, res_ref, mm_ref, wvec_ref):
    i = pl.program_id(0)
    dn = (((1,), (1,)), ((), ()))
    f32 = jnp.float32
    bf16 = jnp.bfloat16

    @pl.when(i == 0)
    def _prep():
        Z = z_ref[...]
        jc = jax.lax.broadcasted_iota(jnp.int32, (L, L), 0).astype(f32)
        lc = jax.lax.broadcasted_iota(jnp.int32, (L, L), 1).astype(f32)
        band = (jnp.abs(jc - lc) <= PAD).astype(f32)
        front = jnp.where(lc == 0, jnp.maximum(PAD - jc, 0.0), 0.0)
        back = jnp.where(lc == L - 1, jnp.maximum(jc - (L - 1 - PAD), 0.0), 0.0)
        A = (band + front + back) * (1.0 / K)
        mm = jax.lax.dot_general(Z, A, dn, preferred_element_type=f32)
        mm_ref[...] = mm
        res_ref[...] = Z - mm
        h = jnp.maximum(
            jax.lax.dot_general(comb_ref[...], rw1_ref[...], dn,
                                preferred_element_type=f32) + rb1_ref[...],
            0.0)
        logits = jax.lax.dot_general(h, rw2_ref[...], dn,
                                     preferred_element_type=f32) + rb2_ref[...]
        mx = jnp.max(logits, axis=1, keepdims=True)
        e = jnp.exp(logits - mx)
        sm = e / jnp.sum(e, axis=1, keepdims=True) * (1.0 - UW)
        wvec_ref[...] = jnp.concatenate(
            [jnp.full((B, 1), UW, f32), sm[:, :NE - 1]], axis=1)

    dec = (jax.lax.dot_general(res_ref[...], ws_ref[0], dn,
                               preferred_element_type=f32)
           + jax.lax.dot_general(mm_ref[...], wt_ref[0], dn,
                                 preferred_element_type=f32)
           + bst_ref[0])
    h = jnp.maximum(
        jax.lax.dot_general(dec, w1_ref[0], dn, preferred_element_type=f32)
        + b1_ref[0], 0.0)
    lane = jax.lax.broadcasted_iota(jnp.int32, (1, NE), 1)
    w = jnp.sum(wvec_ref[...] * (lane == i).astype(f32), axis=1, keepdims=True)
    g = w * h
    o = jax.lax.dot_general(g, w2_ref[0], dn, preferred_element_type=f32)

    @pl.when(i == 0)
    def _():
        out_ref[...] = o

    @pl.when(i > 0)
    def _():
        out_ref[...] += o

    @pl.when(i == NE - 1)
    def _():
        out_ref[...] += jax.lax.dot_general(
            wvec_ref[...], b2_ref[...], (((1,), (0,)), ((), ())),
            preferred_element_type=f32)


@functools.partial(jax.jit, static_argnames=())
def kernel(z, cov_embedding, wearable_embedding, expert_Ws, expert_bs,
           expert_Wt, expert_bt, expert_W1, expert_b1, expert_W2, expert_b2,
           router_W1, router_b1, router_W2, router_b2):
    zsq = z[:, :, 0]
    comb = jnp.concatenate([cov_embedding, wearable_embedding], axis=1)
    rb1 = router_b1.reshape(1, HID)
    rw2 = jnp.concatenate([router_W2, jnp.zeros((1, HID), jnp.float32)], axis=0)
    rb2 = jnp.concatenate([router_b2, jnp.full((1,), -1e9, jnp.float32)]
                          ).reshape(1, NE)
    bst = (expert_bs + expert_bt).reshape(NE, 1, P)
    b1r = expert_b1.reshape(NE, 1, HID)

    out = pl.pallas_call(
        _moe_kernel,
        grid=(NE,),
        in_specs=[
            pl.BlockSpec((B, L), lambda i: (0, 0)),
            pl.BlockSpec((B, 2 * HID), lambda i: (0, 0)),
            pl.BlockSpec((HID, 2 * HID), lambda i: (0, 0)),
            pl.BlockSpec((1, HID), lambda i: (0, 0)),
            pl.BlockSpec((NE, HID), lambda i: (0, 0)),
            pl.BlockSpec((1, NE), lambda i: (0, 0)),
            pl.BlockSpec((1, 1, P), lambda i: (i, 0, 0)),
            pl.BlockSpec((1, 1, HID), lambda i: (i, 0, 0)),
            pl.BlockSpec((NE, P), lambda i: (0, 0)),
            pl.BlockSpec((1, P, L), lambda i: (i, 0, 0)),
            pl.BlockSpec((1, P, L), lambda i: (i, 0, 0)),
            pl.BlockSpec((1, HID, P), lambda i: (i, 0, 0)),
            pl.BlockSpec((1, P, HID), lambda i: (i, 0, 0)),
        ],
        out_specs=pl.BlockSpec((B, P), lambda i: (0, 0)),
        out_shape=jax.ShapeDtypeStruct((B, P), jnp.float32),
        scratch_shapes=[
            pltpu.VMEM((B, L), jnp.float32),
            pltpu.VMEM((B, L), jnp.float32),
            pltpu.VMEM((B, NE), jnp.float32),
        ],
    )(zsq, comb, router_W1, rb1, rw2, rb2, bst, b1r, expert_b2,
      expert_Ws, expert_Wt, expert_W1, expert_W2)

    return out[..., None]
